# baseline (speedup 1.0000x reference)
import functools

import jax
import jax.numpy as jnp
from jax import lax
from jax.experimental import pallas as pl
from jax.experimental.pallas import tpu as pltpu
from jax.experimental.pallas import tpu_sc as plsc

NC = 2
NS = 16
NT = NC * NS
NO = 8
OW = 8
NP = NO // NC
CHUNK = 256


def _round_up(v, m):
    return ((v + m - 1) // m) * m


def _make_deg_kernel(npad, ept32):
    mesh = plsc.VectorSubcoreMesh(
        core_axis_name="c", subcore_axis_name="s", num_cores=NC,
        num_subcores=NS)

    @functools.partial(
        pl.kernel,
        out_type=jax.ShapeDtypeStruct((NT, npad), jnp.float32),
        mesh=mesh,
        compiler_params=pltpu.CompilerParams(
            needs_layout_passes=False, use_tc_tiling_on_sc=False),
        scratch_types=[
            pltpu.VMEM((ept32,), jnp.int32),
            pltpu.VMEM((npad,), jnp.float32),
        ],
    )
    def deg_kernel(col_hbm, out_hbm, col_v, acc_v):
        c = lax.axis_index("c")
        s = lax.axis_index("s")
        w = s * NC + c
        pltpu.sync_copy(col_hbm.at[w], col_v)
        zeros = jnp.zeros((16,), jnp.float32)

        def zloop(k, _):
            acc_v[pl.ds(k * 16, 16)] = zeros
            return 0

        lax.fori_loop(0, npad // 16, zloop, 0)
        ones = jnp.ones((16,), jnp.float32)

        def cloop(k, _):
            idx = col_v[pl.ds(k * 16, 16)]
            plsc.addupdate_scatter(acc_v, [idx], ones)
            return 0

        lax.fori_loop(0, ept32 // 16, cloop, 0)
        pltpu.sync_copy(acc_v, out_hbm.at[w])

    return deg_kernel


def _make_agg_kernel(npad, nr, ept16, ch):
    rpt = nr // NS
    zch = rpt // 128
    zrem = rpt % 128
    mesh = plsc.VectorSubcoreMesh(
        core_axis_name="c", subcore_axis_name="s", num_cores=NC,
        num_subcores=NS)

    @functools.partial(
        pl.kernel,
        out_type=jax.ShapeDtypeStruct((NO * npad, OW), jnp.float32),
        mesh=mesh,
        compiler_params=pltpu.CompilerParams(
            needs_layout_passes=False, use_tc_tiling_on_sc=False),
        scratch_types=[
            pltpu.VMEM((ept16,), jnp.int32),
            pltpu.VMEM((2 * ch, 128), jnp.int32),
            pltpu.VMEM((2, CHUNK, OW), jnp.float32),
            pltpu.VMEM((128, OW), jnp.float32),
            pltpu.VMEM_SHARED((nr, OW), jnp.float32),
            pltpu.SemaphoreType.DMA((2,)),
        ],
    )
    def agg_kernel(tp_hbm, row_hbm, col_hbm, zeros_hbm, out_hbm,
                   row_v, col_v, msg_v, zrow_v, acc_sh, gsem):
        c = lax.axis_index("c")
        s = lax.axis_index("s")
        pltpu.sync_copy(row_hbm.at[s], row_v)
        pltpu.sync_copy(col_hbm.at[s], col_v)
        pltpu.sync_copy(zeros_hbm, zrow_v)
        base = c * (NP * npad)

        def adj0(k, _):
            row_v[pl.ds(k * 16, 16)] = row_v[pl.ds(k * 16, 16)] * NP + base
            return 0

        def adj1(k, _):
            row_v[pl.ds(k * 16, 16)] = row_v[pl.ds(k * 16, 16)] + 1
            return 0

        lax.fori_loop(0, ept16 // 16, adj0, 0)

        def z2(k, _):
            pltpu.sync_copy(zrow_v, acc_sh.at[pl.ds(s * rpt + k * 128, 128)])
            return 0

        def g_start(j, b):
            pltpu.async_copy(
                tp_hbm.at[row_v.at[pl.ds(j * CHUNK, CHUNK)]],
                msg_v.at[b], gsem.at[b])

        def g_wait(j, b):
            pltpu.make_async_copy(
                tp_hbm.at[row_v.at[pl.ds(j * CHUNK, CHUNK)]],
                msg_v.at[b], gsem.at[b]).wait()

        def ring(k, _):
            for u in range(2):
                j = k * 2 + u

                @pl.when(j + 1 < ch)
                def _(j=j):
                    g_start(j + 1, (j + 1) % 2)

                @pl.when(j < ch)
                def _(j=j, u=u):
                    g_wait(j, u)
                    for m in range(2):
                        pltpu.sync_copy(
                            msg_v.at[u, pl.ds(m * 128, 128)],
                            acc_sh.at[col_v.at[2 * j + m]], add=True)
            return 0

        for p in range(NP):
            if p:
                lax.fori_loop(0, ept16 // 16, adj1, 0)
            lax.fori_loop(0, zch, z2, 0)
            if zrem:
                pltpu.sync_copy(
                    zrow_v.at[pl.ds(0, zrem)],
                    acc_sh.at[pl.ds(s * rpt + zch * 128, zrem)])
            plsc.subcore_barrier()
            for b in range(min(1, ch)):
                g_start(b, b)
            lax.fori_loop(0, (ch + 1) // 2, ring, 0)
            plsc.subcore_barrier()
            pltpu.sync_copy(
                acc_sh.at[pl.ds(s * rpt, rpt)],
                out_hbm.at[pl.ds((c * NP + p) * npad + s * rpt, rpt)])

    return agg_kernel


def _cat_s(s_ref):
    return jnp.concatenate([s_ref[o] for o in range(NO)], axis=1)


def _cat_tp(tp_ref):
    return jnp.concatenate([tp_ref[0], tp_ref[1]], axis=1)


def _store_tp(t, tp_ref):
    tp_ref[0] = t[:, :32]
    tp_ref[1] = t[:, 32:]


def _a_body(n, bn, x_ref, deg_ref, wemb_ref, bemb_ref, wc_ref, dinv_ref,
            tp_ref):
    xb = x_ref[...]
    h0 = jnp.dot(xb, wemb_ref[...],
                 preferred_element_type=jnp.float32) + bemb_ref[...]
    deg_col = lax.dot_general(
        deg_ref[...], jnp.ones((NT, 1), jnp.float32),
        (((0,), (0,)), ((), ())), preferred_element_type=jnp.float32)
    dinv = lax.rsqrt(deg_col + 1.0)
    dinv_ref[...] = dinv
    t = jnp.dot(h0, wc_ref[...], preferred_element_type=jnp.float32) * dinv
    rows = pl.program_id(0) * bn + lax.broadcasted_iota(jnp.int32, (bn, 1), 0)
    _store_tp(jnp.where(rows < n, t, 0.0), tp_ref)


def _b_body(n, bn, s_ref, tp_ref, dinv_ref, bc_ref, wc_ref, out_ref):
    s64 = _cat_s(s_ref)
    t64 = _cat_tp(tp_ref)
    dinv = dinv_ref[...]
    h = jnp.maximum(dinv * (s64 + t64) + bc_ref[...], 0.0)
    t = jnp.dot(h, wc_ref[...], preferred_element_type=jnp.float32) * dinv
    rows = pl.program_id(0) * bn + lax.broadcasted_iota(jnp.int32, (bn, 1), 0)
    _store_tp(jnp.where(rows < n, t, 0.0), out_ref)


def _c_body(n, bn, s_ref, tp_ref, dinv_ref, bc_ref, batch_ref, w1_ref,
            b1_ref, w2_ref, b2_ref, pred_ref, sums_ref, cnt_ref):
    i = pl.program_id(0)
    s64 = _cat_s(s_ref)
    t64 = _cat_tp(tp_ref)
    dinv = dinv_ref[...]
    h = jnp.maximum(dinv * (s64 + t64) + bc_ref[...], 0.0)
    rows = i * bn + lax.broadcasted_iota(jnp.int32, (bn, 1), 0)
    valid = rows < n
    hm = jnp.where(valid, h, 0.0)
    bcol = batch_ref[0]
    iota_g = lax.broadcasted_iota(jnp.int32, (bn, 128), 1)
    oh = jnp.where((bcol == iota_g) & valid, 1.0, 0.0)

    @pl.when(i == 0)
    def _():
        sums_ref[...] = jnp.zeros_like(sums_ref)
        cnt_ref[...] = jnp.zeros_like(cnt_ref)

    sums_ref[...] += lax.dot_general(
        oh, hm, (((0,), (0,)), ((), ())), preferred_element_type=jnp.float32)
    cnt_ref[...] += lax.dot_general(
        oh, jnp.ones((bn, 8), jnp.float32), (((0,), (0,)), ((), ())),
        preferred_element_type=jnp.float32)
    graph = sums_ref[...] / jnp.maximum(cnt_ref[...][:, 0:1], 1.0)
    p = jnp.maximum(
        jnp.dot(graph, w1_ref[...], preferred_element_type=jnp.float32)
        + b1_ref[...], 0.0)
    pred_ref[...] = jnp.dot(
        p, w2_ref[...], preferred_element_type=jnp.float32) + b2_ref[...]


def kernel(x, edge_index, batch, W_emb, b_emb, Wc, bc, W1, b1, W2, b2):
    n = x.shape[0]
    e = edge_index.shape[1]
    h = W_emb.shape[1]
    num_unit = Wc.shape[0]
    npad = _round_up(n + 1, 2048)
    e_pad = _round_up(e, NS * CHUNK)
    ept16 = e_pad // NS
    ch = ept16 // CHUNK
    ept32 = e_pad // NT
    bn = 1024
    nblk = npad // bn

    row = edge_index[0]
    col = edge_index[1]
    pad = e_pad - e
    rowp = jnp.concatenate(
        [row, jnp.full((pad,), n, jnp.int32)]).astype(jnp.int32)
    colp = jnp.concatenate(
        [col, jnp.full((pad,), n, jnp.int32)]).astype(jnp.int32)
    col32 = colp.reshape(NT, ept32)
    row16 = rowp.reshape(NS, ept16)
    col16 = colp.reshape(NS, 2 * ch, 128)
    zeros_sc = jnp.zeros((128, OW), jnp.float32)
    xp = jnp.concatenate([x, jnp.zeros((npad - n, x.shape[1]), x.dtype)])
    batch3 = jnp.concatenate(
        [batch.astype(jnp.int32), jnp.zeros((npad - n,), jnp.int32)]
    ).reshape(nblk, bn, 1)

    bemb2 = b_emb.reshape(1, h)
    bc2 = bc.reshape(num_unit, 1, h)
    b12 = b1.reshape(1, h)
    b22 = b2.reshape(1, W2.shape[1])

    deg_kernel = _make_deg_kernel(npad, ept32)
    nr = _round_up(n + 1, NS)
    agg_kernel = _make_agg_kernel(npad, nr, ept16, ch)

    deg_part = deg_kernel(col32)

    full = lambda shp: pl.BlockSpec(shp, lambda i: tuple(0 for _ in shp))
    tp_spec = pl.BlockSpec((NC, bn, 32), lambda i: (0, i, 0))
    s_spec = pl.BlockSpec((NO, bn, OW), lambda i: (0, i, 0))
    dinv_spec = pl.BlockSpec((bn, 1), lambda i: (i, 0))

    dinv, tp = pl.pallas_call(
        functools.partial(_a_body, n, bn),
        grid=(nblk,),
        in_specs=[
            pl.BlockSpec((bn, 4), lambda i: (i, 0)),
            pl.BlockSpec((NT, bn), lambda i: (0, i)),
            full((4, h)),
            full((1, h)),
            full((h, h)),
        ],
        out_specs=[dinv_spec, tp_spec],
        out_shape=[
            jax.ShapeDtypeStruct((npad, 1), jnp.float32),
            jax.ShapeDtypeStruct((NC, npad, 32), jnp.float32),
        ],
    )(xp, deg_part, W_emb, bemb2, Wc[0])

    wc_shift = jnp.concatenate([Wc[1:], Wc[:1]], axis=0)
    trip = num_unit + lax.shift_right_arithmetic(row[0], 31)

    def layer_cond(carry):
        return carry[0] < trip

    def layer_step(carry):
        i, tp_c, _, _ = carry
        wc_n = lax.dynamic_index_in_dim(wc_shift, i, keepdims=False)
        bc_i = lax.dynamic_index_in_dim(bc2, i, keepdims=False)
        s3 = agg_kernel(
            tp_c.reshape(NO * npad, OW), row16, col16, zeros_sc
        ).reshape(NO, npad, OW)
        tp_n = pl.pallas_call(
            functools.partial(_b_body, n, bn),
            grid=(nblk,),
            in_specs=[
                s_spec,
                tp_spec,
                dinv_spec,
                full((1, h)),
                full((h, h)),
            ],
            out_specs=tp_spec,
            out_shape=jax.ShapeDtypeStruct((NC, npad, 32), jnp.float32),
        )(s3, tp_c, dinv, bc_i, wc_n)
        return (i + 1, tp_n, tp_c, s3)

    init = (jnp.int32(0), tp, jnp.zeros_like(tp),
            jnp.zeros((NO, npad, OW), jnp.float32))
    _, _, tp, s_agg3 = lax.while_loop(layer_cond, layer_step, init)

    pred = pl.pallas_call(
        functools.partial(_c_body, n, bn),
        grid=(nblk,),
        in_specs=[
            s_spec,
            tp_spec,
            dinv_spec,
            full((1, h)),
            pl.BlockSpec((1, bn, 1), lambda i: (i, 0, 0)),
            full((h, h)),
            full((1, h)),
            full((h, W2.shape[1])),
            full((1, W2.shape[1])),
        ],
        out_specs=pl.BlockSpec((128, W2.shape[1]), lambda i: (0, 0)),
        out_shape=jax.ShapeDtypeStruct((128, W2.shape[1]), jnp.float32),
        scratch_shapes=[
            pltpu.VMEM((128, h), jnp.float32),
            pltpu.VMEM((128, 8), jnp.float32),
        ],
    )(s_agg3, tp, dinv, bc2[num_unit - 1], batch3,
      W1, b12, W2, b22)
    return pred


# speedup vs baseline: 1.1223x; 1.1223x over previous
import functools

import jax
import jax.numpy as jnp
from jax import lax
from jax.experimental import pallas as pl
from jax.experimental.pallas import tpu as pltpu
from jax.experimental.pallas import tpu_sc as plsc

NC = 2
NS = 16
NT = NC * NS
NO = 8
OW = 8
NP = NO // NC


def _round_up(v, m):
    return ((v + m - 1) // m) * m


def _make_deg_kernel(npad, ept32):
    mesh = plsc.VectorSubcoreMesh(
        core_axis_name="c", subcore_axis_name="s", num_cores=NC,
        num_subcores=NS)

    @functools.partial(
        pl.kernel,
        out_type=jax.ShapeDtypeStruct((NT, npad), jnp.float32),
        mesh=mesh,
        compiler_params=pltpu.CompilerParams(
            needs_layout_passes=False, use_tc_tiling_on_sc=False),
        scratch_types=[
            pltpu.VMEM((ept32,), jnp.int32),
            pltpu.VMEM((npad,), jnp.float32),
        ],
    )
    def deg_kernel(col_hbm, out_hbm, col_v, acc_v):
        c = lax.axis_index("c")
        s = lax.axis_index("s")
        w = s * NC + c
        pltpu.sync_copy(col_hbm.at[w], col_v)
        zeros = jnp.zeros((16,), jnp.float32)

        def zloop(k, _):
            acc_v[pl.ds(k * 16, 16)] = zeros
            return 0

        lax.fori_loop(0, npad // 16, zloop, 0)
        ones = jnp.ones((16,), jnp.float32)

        def cloop(k, _):
            idx = col_v[pl.ds(k * 16, 16)]
            plsc.addupdate_scatter(acc_v, [idx], ones)
            return 0

        lax.fori_loop(0, ept32 // 16, cloop, 0)
        pltpu.sync_copy(acc_v, out_hbm.at[w])

    return deg_kernel


def _make_agg_kernel(npad, ept16, ch):
    rpt = npad // NS
    zch = rpt // 128
    mesh = plsc.VectorSubcoreMesh(
        core_axis_name="c", subcore_axis_name="s", num_cores=NC,
        num_subcores=NS)

    @functools.partial(
        pl.kernel,
        out_type=jax.ShapeDtypeStruct((NO * npad, OW), jnp.float32),
        mesh=mesh,
        compiler_params=pltpu.CompilerParams(
            needs_layout_passes=False, use_tc_tiling_on_sc=False),
        scratch_types=[
            pltpu.VMEM((ept16,), jnp.int32),
            pltpu.VMEM((ch, 128), jnp.int32),
            pltpu.VMEM((4, 128, OW), jnp.float32),
            pltpu.VMEM((128, OW), jnp.float32),
            pltpu.VMEM_SHARED((npad, OW), jnp.float32),
            pltpu.SemaphoreType.DMA((4,)),
        ],
    )
    def agg_kernel(tp_hbm, row_hbm, col_hbm, zeros_hbm, out_hbm,
                   row_v, col_v, msg_v, zrow_v, acc_sh, gsem):
        c = lax.axis_index("c")
        s = lax.axis_index("s")
        pltpu.sync_copy(row_hbm.at[s], row_v)
        pltpu.sync_copy(col_hbm.at[s], col_v)
        pltpu.sync_copy(zeros_hbm, zrow_v)
        base = c * (NP * npad)

        def adj0(k, _):
            row_v[pl.ds(k * 16, 16)] = row_v[pl.ds(k * 16, 16)] * NP + base
            return 0

        def adj1(k, _):
            row_v[pl.ds(k * 16, 16)] = row_v[pl.ds(k * 16, 16)] + 1
            return 0

        lax.fori_loop(0, ept16 // 16, adj0, 0)

        def z2(k, _):
            pltpu.sync_copy(zrow_v, acc_sh.at[pl.ds(s * rpt + k * 128, 128)])
            return 0

        def g_start(j, b):
            pltpu.async_copy(
                tp_hbm.at[row_v.at[pl.ds(j * 128, 128)]],
                msg_v.at[b], gsem.at[b])

        def g_wait(j, b):
            pltpu.make_async_copy(
                tp_hbm.at[row_v.at[pl.ds(j * 128, 128)]],
                msg_v.at[b], gsem.at[b]).wait()

        def ring(k, _):
            for u in range(4):
                j = k * 4 + u

                @pl.when(j + 3 < ch)
                def _(j=j):
                    g_start(j + 3, (j + 3) % 4)

                @pl.when(j < ch)
                def _(j=j, u=u):
                    g_wait(j, u)
                    pltpu.sync_copy(
                        msg_v.at[u], acc_sh.at[col_v.at[j]], add=True)
            return 0

        for p in range(NP):
            if p:
                lax.fori_loop(0, ept16 // 16, adj1, 0)
            lax.fori_loop(0, zch, z2, 0)
            plsc.subcore_barrier()
            for b in range(min(3, ch)):
                g_start(b, b)
            lax.fori_loop(0, (ch + 3) // 4, ring, 0)
            plsc.subcore_barrier()
            pltpu.sync_copy(
                acc_sh.at[pl.ds(s * rpt, rpt)],
                out_hbm.at[pl.ds((c * NP + p) * npad + s * rpt, rpt)])

    return agg_kernel


def _cat_s(s_ref):
    return jnp.concatenate([s_ref[o] for o in range(NO)], axis=1)


def _cat_tp(tp_ref):
    return jnp.concatenate([tp_ref[0], tp_ref[1]], axis=1)


def _store_tp(t, tp_ref):
    tp_ref[0] = t[:, :32]
    tp_ref[1] = t[:, 32:]


def _a_body(n, bn, x_ref, deg_ref, wemb_ref, bemb_ref, wc_ref, dinv_ref,
            tp_ref):
    xb = x_ref[...]
    h0 = jnp.dot(xb, wemb_ref[...],
                 preferred_element_type=jnp.float32) + bemb_ref[...]
    deg_col = lax.dot_general(
        deg_ref[...], jnp.ones((NT, 1), jnp.float32),
        (((0,), (0,)), ((), ())), preferred_element_type=jnp.float32)
    dinv = lax.rsqrt(deg_col + 1.0)
    dinv_ref[...] = dinv
    t = jnp.dot(h0, wc_ref[...], preferred_element_type=jnp.float32) * dinv
    rows = pl.program_id(0) * bn + lax.broadcasted_iota(jnp.int32, (bn, 1), 0)
    _store_tp(jnp.where(rows < n, t, 0.0), tp_ref)


def _b_body(n, bn, s_ref, tp_ref, dinv_ref, bc_ref, wc_ref, out_ref):
    s64 = _cat_s(s_ref)
    t64 = _cat_tp(tp_ref)
    dinv = dinv_ref[...]
    h = jnp.maximum(dinv * (s64 + t64) + bc_ref[...], 0.0)
    t = jnp.dot(h, wc_ref[...], preferred_element_type=jnp.float32) * dinv
    rows = pl.program_id(0) * bn + lax.broadcasted_iota(jnp.int32, (bn, 1), 0)
    _store_tp(jnp.where(rows < n, t, 0.0), out_ref)


def _c_body(n, bn, s_ref, tp_ref, dinv_ref, bc_ref, batch_ref, w1_ref,
            b1_ref, w2_ref, b2_ref, pred_ref, sums_ref, cnt_ref):
    i = pl.program_id(0)
    s64 = _cat_s(s_ref)
    t64 = _cat_tp(tp_ref)
    dinv = dinv_ref[...]
    h = jnp.maximum(dinv * (s64 + t64) + bc_ref[...], 0.0)
    rows = i * bn + lax.broadcasted_iota(jnp.int32, (bn, 1), 0)
    valid = rows < n
    hm = jnp.where(valid, h, 0.0)
    bcol = batch_ref[0]
    iota_g = lax.broadcasted_iota(jnp.int32, (bn, 128), 1)
    oh = jnp.where((bcol == iota_g) & valid, 1.0, 0.0)

    @pl.when(i == 0)
    def _():
        sums_ref[...] = jnp.zeros_like(sums_ref)
        cnt_ref[...] = jnp.zeros_like(cnt_ref)

    sums_ref[...] += lax.dot_general(
        oh, hm, (((0,), (0,)), ((), ())), preferred_element_type=jnp.float32)
    cnt_ref[...] += lax.dot_general(
        oh, jnp.ones((bn, 8), jnp.float32), (((0,), (0,)), ((), ())),
        preferred_element_type=jnp.float32)
    graph = sums_ref[...] / jnp.maximum(cnt_ref[...][:, 0:1], 1.0)
    p = jnp.maximum(
        jnp.dot(graph, w1_ref[...], preferred_element_type=jnp.float32)
        + b1_ref[...], 0.0)
    pred_ref[...] = jnp.dot(
        p, w2_ref[...], preferred_element_type=jnp.float32) + b2_ref[...]


def kernel(x, edge_index, batch, W_emb, b_emb, Wc, bc, W1, b1, W2, b2):
    n = x.shape[0]
    e = edge_index.shape[1]
    h = W_emb.shape[1]
    num_unit = Wc.shape[0]
    npad = _round_up(n + 1, 2048)
    e_pad = _round_up(e, 2048)
    ept16 = e_pad // NS
    ch = ept16 // 128
    ept32 = e_pad // NT
    bn = 1024
    nblk = npad // bn

    row = edge_index[0]
    col = edge_index[1]
    pad = e_pad - e
    rowp = jnp.concatenate(
        [row, jnp.full((pad,), n, jnp.int32)]).astype(jnp.int32)
    colp = jnp.concatenate(
        [col, jnp.full((pad,), n, jnp.int32)]).astype(jnp.int32)
    col32 = colp.reshape(NT, ept32)
    row16 = rowp.reshape(NS, ept16)
    col16 = colp.reshape(NS, ch, 128)
    zeros_sc = jnp.zeros((128, OW), jnp.float32)
    xp = jnp.concatenate([x, jnp.zeros((npad - n, x.shape[1]), x.dtype)])
    batch3 = jnp.concatenate(
        [batch.astype(jnp.int32), jnp.zeros((npad - n,), jnp.int32)]
    ).reshape(nblk, bn, 1)

    bemb2 = b_emb.reshape(1, h)
    bc2 = bc.reshape(num_unit, 1, h)
    b12 = b1.reshape(1, h)
    b22 = b2.reshape(1, W2.shape[1])

    deg_kernel = _make_deg_kernel(npad, ept32)
    agg_kernel = _make_agg_kernel(npad, ept16, ch)

    deg_part = deg_kernel(col32)

    full = lambda shp: pl.BlockSpec(shp, lambda i: tuple(0 for _ in shp))
    tp_spec = pl.BlockSpec((NC, bn, 32), lambda i: (0, i, 0))
    s_spec = pl.BlockSpec((NO, bn, OW), lambda i: (0, i, 0))
    dinv_spec = pl.BlockSpec((bn, 1), lambda i: (i, 0))

    dinv, tp = pl.pallas_call(
        functools.partial(_a_body, n, bn),
        grid=(nblk,),
        in_specs=[
            pl.BlockSpec((bn, 4), lambda i: (i, 0)),
            pl.BlockSpec((NT, bn), lambda i: (0, i)),
            full((4, h)),
            full((1, h)),
            full((h, h)),
        ],
        out_specs=[dinv_spec, tp_spec],
        out_shape=[
            jax.ShapeDtypeStruct((npad, 1), jnp.float32),
            jax.ShapeDtypeStruct((NC, npad, 32), jnp.float32),
        ],
    )(xp, deg_part, W_emb, bemb2, Wc[0])

    wc_shift = jnp.concatenate([Wc[1:], Wc[:1]], axis=0)
    trip = num_unit + lax.shift_right_arithmetic(row[0], 31)

    def layer_cond(carry):
        return carry[0] < trip

    def layer_step(carry):
        i, tp_c, _, _ = carry
        wc_n = lax.dynamic_index_in_dim(wc_shift, i, keepdims=False)
        bc_i = lax.dynamic_index_in_dim(bc2, i, keepdims=False)
        s3 = agg_kernel(
            tp_c.reshape(NO * npad, OW), row16, col16, zeros_sc
        ).reshape(NO, npad, OW)
        tp_n = pl.pallas_call(
            functools.partial(_b_body, n, bn),
            grid=(nblk,),
            in_specs=[
                s_spec,
                tp_spec,
                dinv_spec,
                full((1, h)),
                full((h, h)),
            ],
            out_specs=tp_spec,
            out_shape=jax.ShapeDtypeStruct((NC, npad, 32), jnp.float32),
        )(s3, tp_c, dinv, bc_i, wc_n)
        return (i + 1, tp_n, tp_c, s3)

    init = (jnp.int32(0), tp, jnp.zeros_like(tp),
            jnp.zeros((NO, npad, OW), jnp.float32))
    _, _, tp, s_agg3 = lax.while_loop(layer_cond, layer_step, init)

    pred = pl.pallas_call(
        functools.partial(_c_body, n, bn),
        grid=(nblk,),
        in_specs=[
            s_spec,
            tp_spec,
            dinv_spec,
            full((1, h)),
            pl.BlockSpec((1, bn, 1), lambda i: (i, 0, 0)),
            full((h, h)),
            full((1, h)),
            full((h, W2.shape[1])),
            full((1, W2.shape[1])),
        ],
        out_specs=pl.BlockSpec((128, W2.shape[1]), lambda i: (0, 0)),
        out_shape=jax.ShapeDtypeStruct((128, W2.shape[1]), jnp.float32),
        scratch_shapes=[
            pltpu.VMEM((128, h), jnp.float32),
            pltpu.VMEM((128, 8), jnp.float32),
        ],
    )(s_agg3, tp, dinv, bc2[num_unit - 1], batch3,
      W1, b12, W2, b22)
    return pred
